# Initial kernel scaffold; baseline (speedup 1.0000x reference)
import jax
import jax.numpy as jnp
from jax.experimental import pallas as pl


def kernel(x):
    raise NotImplementedError("write your pallas kernel here")


# speedup vs baseline: 7.5582x; 7.5582x over previous
import jax
import jax.numpy as jnp
from jax.experimental import pallas as pl


def _corner_kernel(x_ref, o_ref):
    v = x_ref[...]
    cb, h, w = v.shape
    k = 1
    while k < h:
        pad = jnp.full((cb, k, w), -jnp.inf, v.dtype)
        v = jnp.maximum(v, jnp.concatenate([pad, v[:, :-k, :]], axis=1))
        k *= 2
    k = 1
    while k < w:
        pad = jnp.full((cb, h, k), -jnp.inf, v.dtype)
        v = jnp.maximum(v, jnp.concatenate([pad, v[:, :, :-k]], axis=2))
        k *= 2
    o_ref[...] = v + v


def kernel(x):
    b, c, h, w = x.shape
    xf = x.reshape(b * c, h, w)
    cb = 8
    out = pl.pallas_call(
        _corner_kernel,
        grid=((b * c) // cb,),
        in_specs=[pl.BlockSpec((cb, h, w), lambda i: (i, 0, 0))],
        out_specs=pl.BlockSpec((cb, h, w), lambda i: (i, 0, 0)),
        out_shape=jax.ShapeDtypeStruct((b * c, h, w), x.dtype),
    )(xf)
    return out.reshape(b, c, h, w)
